# Initial kernel scaffold; baseline (speedup 1.0000x reference)
import jax
import jax.numpy as jnp
from jax.experimental import pallas as pl


def kernel(x, table, W, b):
    raise NotImplementedError("write your pallas kernel here")


# speedup vs baseline: 33.2527x; 33.2527x over previous
import jax
import jax.numpy as jnp
from jax import lax
from jax.experimental import pallas as pl
from jax.experimental.pallas import tpu as pltpu
from jax.experimental.pallas import tpu_sc as plsc

_B = 16384
_SEQ = 50
_D = 32
_NCLS = 2
_NC = 2
_NS = 16
_NW = _NC * _NS
_RPW = _B // _NW
_CB = 16
_NCHUNK = _RPW // _CB
_GROWS = _CB * _SEQ
_IPW = _RPW * _SEQ


def _sc_body(x_ref, tab_ref, w_ref, out_ref, idx_v, emb_v, w_v, out_v, red_v, sem):
    wid = lax.axis_index("s") * _NC + lax.axis_index("c")
    pltpu.sync_copy(x_ref.at[pl.ds(wid * _IPW, _IPW)], idx_v)
    pltpu.sync_copy(w_ref, w_v)
    lanes = jnp.arange(16, dtype=jnp.int32)

    def do_chunk(c, _):
        pltpu.async_copy(
            tab_ref.at[idx_v.at[pl.ds(c * _GROWS, _GROWS)]], emb_v, sem
        ).wait()

        def s_body(s, accs):
            a0, a1 = accs
            o = s * _D
            w0l = w_v[0, pl.ds(o, 16)]
            w0h = w_v[0, pl.ds(o + 16, 16)]
            w1l = w_v[1, pl.ds(o, 16)]
            w1h = w_v[1, pl.ds(o + 16, 16)]
            na0, na1 = [], []
            for r in range(_CB):
                el = emb_v[r * _SEQ + s, pl.ds(0, 16)]
                eh = emb_v[r * _SEQ + s, pl.ds(16, 16)]
                na0.append(a0[r] + el * w0l + eh * w0h)
                na1.append(a1[r] + el * w1l + eh * w1h)
            return na0, na1

        zero = jnp.zeros((16,), jnp.float32)
        a0, a1 = lax.fori_loop(
            0, _SEQ, s_body, ([zero] * _CB, [zero] * _CB)
        )
        base = lanes * 16
        for cls, acc in ((0, a0), (1, a1)):
            for r in range(_CB):
                red_v[pl.ds(r * 16, 16)] = acc[r]
            tot = plsc.load_gather(red_v, [base])
            for j in range(1, 16):
                tot = tot + plsc.load_gather(red_v, [base + j])
            out_v[cls, pl.ds(c * _CB, _CB)] = tot
        return ()

    lax.fori_loop(0, _NCHUNK, do_chunk, ())
    pltpu.sync_copy(out_v, out_ref.at[wid])


@jax.jit
def _run(x_flat, table, w):
    mesh = plsc.VectorSubcoreMesh(core_axis_name="c", subcore_axis_name="s")
    f = pl.kernel(
        _sc_body,
        out_type=jax.ShapeDtypeStruct((_NW, _NCLS, _RPW), jnp.float32),
        mesh=mesh,
        scratch_types=[
            pltpu.VMEM((_IPW,), jnp.int32),
            pltpu.VMEM((_GROWS, _D), jnp.float32),
            pltpu.VMEM((_NCLS, _SEQ * _D), jnp.float32),
            pltpu.VMEM((_NCLS, _RPW), jnp.float32),
            pltpu.VMEM((256,), jnp.float32),
            pltpu.SemaphoreType.DMA,
        ],
        compiler_params=pltpu.CompilerParams(
            needs_layout_passes=False, use_tc_tiling_on_sc=False
        ),
    )
    return f(x_flat, table, w)


def kernel(x, table, W, b):
    out = _run(x.reshape(-1).astype(jnp.int32), table, W)
    return out.transpose(0, 2, 1).reshape(_B, _NCLS) + b


# speedup vs baseline: 34.3167x; 1.0320x over previous
import jax
import jax.numpy as jnp
from jax import lax
from jax.experimental import pallas as pl
from jax.experimental.pallas import tpu as pltpu
from jax.experimental.pallas import tpu_sc as plsc

_B = 16384
_SEQ = 50
_D = 32
_NCLS = 2
_V = 1_000_000
_NC = 2
_NS = 16
_NW = _NC * _NS
_RPW = _B // _NW
_CB = 16
_NCHUNK = _RPW // _CB
_GROWS = _CB * _SEQ

_TB = 2048
_TGRID = (_V + _TB - 1) // _TB


def _tr_body(in_ref, out_ref):
    in_t = in_ref[...].T.reshape(_TB // 4, 4, _D)
    d_io = lax.broadcasted_iota(jnp.int32, (_D, 128), 0)
    c_io = lax.broadcasted_iota(jnp.int32, (_D, 128), 1)
    acc = None
    for j in range(4):
        shift = (c_io == d_io + _D * j).astype(jnp.float32)
        term = lax.dot_general(
            in_t[:, j, :], shift, (((1,), (0,)), ((), ())),
            preferred_element_type=jnp.float32,
        )
        acc = term if acc is None else acc + term
    out_ref[...] = acc


def _transpose_table(t_t):
    return pl.pallas_call(
        _tr_body,
        grid=(_TGRID,),
        in_specs=[pl.BlockSpec((_D, _TB), lambda k: (0, k))],
        out_specs=pl.BlockSpec((_TB // 4, 128), lambda k: (k, 0)),
        out_shape=jax.ShapeDtypeStruct((_V * _D // 128, 128), jnp.float32),
    )(t_t)


def _sc_body(x_ref, tab_ref, w_ref, out_ref,
             idx_v, idxc_v, emb_v, w_v, out_v, red_v, sem):
    wid = lax.axis_index("s") * _NC + lax.axis_index("c")
    pltpu.sync_copy(x_ref.at[:, pl.ds(wid * _RPW, _RPW)], idx_v)
    pltpu.sync_copy(w_ref, w_v)
    lanes = jnp.arange(16, dtype=jnp.int32)

    def do_chunk(c, _):
        for s in range(_SEQ):
            idxc_v[pl.ds(s * _CB, _CB)] = idx_v[s, pl.ds(c * _CB, _CB)]
        pltpu.async_copy(tab_ref.at[idxc_v], emb_v, sem).wait()

        def s_body(s, accs):
            a0, a1 = accs
            o = s * _D
            w0l = w_v[0, pl.ds(o, 16)]
            w0h = w_v[0, pl.ds(o + 16, 16)]
            w1l = w_v[1, pl.ds(o, 16)]
            w1h = w_v[1, pl.ds(o + 16, 16)]
            na0, na1 = [], []
            for r in range(_CB):
                el = emb_v[s * _CB + r, pl.ds(0, 16)]
                eh = emb_v[s * _CB + r, pl.ds(16, 16)]
                na0.append(a0[r] + el * w0l + eh * w0h)
                na1.append(a1[r] + el * w1l + eh * w1h)
            return na0, na1

        zero = jnp.zeros((16,), jnp.float32)
        a0, a1 = lax.fori_loop(0, _SEQ, s_body, ([zero] * _CB, [zero] * _CB))
        base = lanes * 16
        for cls, acc in ((0, a0), (1, a1)):
            for r in range(_CB):
                red_v[pl.ds(r * 16, 16)] = acc[r]
            tot = plsc.load_gather(red_v, [base])
            for j in range(1, 16):
                tot = tot + plsc.load_gather(red_v, [base + j])
            out_v[cls, pl.ds(c * _CB, _CB)] = tot
        return ()

    lax.fori_loop(0, _NCHUNK, do_chunk, ())
    pltpu.sync_copy(out_v, out_ref.at[wid])


@jax.jit
def _run(x_t, table, w):
    tbl_lin = _transpose_table(table.T).reshape(_V, _D)
    mesh = plsc.VectorSubcoreMesh(core_axis_name="c", subcore_axis_name="s")
    f = pl.kernel(
        _sc_body,
        out_type=jax.ShapeDtypeStruct((_NW, _NCLS, _RPW), jnp.float32),
        mesh=mesh,
        scratch_types=[
            pltpu.VMEM((_SEQ, _RPW), jnp.int32),
            pltpu.VMEM((_GROWS,), jnp.int32),
            pltpu.VMEM((_GROWS, _D), jnp.float32),
            pltpu.VMEM((_NCLS, _SEQ * _D), jnp.float32),
            pltpu.VMEM((_NCLS, _RPW), jnp.float32),
            pltpu.VMEM((256,), jnp.float32),
            pltpu.SemaphoreType.DMA,
        ],
        compiler_params=pltpu.CompilerParams(
            needs_layout_passes=False, use_tc_tiling_on_sc=False
        ),
    )
    return f(x_t, tbl_lin, w)


def kernel(x, table, W, b):
    out = _run(x.T.astype(jnp.int32), table, W)
    return out.transpose(0, 2, 1).reshape(_B, _NCLS) + b
